# baseline (speedup 1.0000x reference)
import functools

import jax
import jax.numpy as jnp
from jax import lax
from jax.experimental import pallas as pl
from jax.experimental.pallas import tpu as pltpu
from jax.experimental.pallas import tpu_sc as plsc

_info = plsc.get_sparse_core_info()
_NC, _NS = _info.num_cores, _info.num_subcores
_NW = _NC * _NS

_BATCH = 16384
_DIM = 32
_B_PER_W = _BATCH // _NW
_CHUNK = 128
_NCHUNK = _B_PER_W // _CHUNK

_mesh = plsc.VectorSubcoreMesh(core_axis_name="c", subcore_axis_name="s")


@functools.partial(
    pl.kernel,
    mesh=_mesh,
    out_type=jax.ShapeDtypeStruct((_BATCH, _DIM), jnp.float32),
    scratch_types=[
        pltpu.VMEM((_NCHUNK, _CHUNK), jnp.int32),
        pltpu.VMEM((_B_PER_W, _DIM), jnp.float32),
        pltpu.SemaphoreType.DMA,
    ],
    compiler_params=pltpu.CompilerParams(use_tc_tiling_on_sc=False),
)
def _embed_gather(idx_hbm, table_hbm, out_hbm, idx_v, rows_v, sem):
    wid = lax.axis_index("s") * _NC + lax.axis_index("c")
    base = wid * _B_PER_W
    pltpu.sync_copy(idx_hbm.at[wid], idx_v)
    copies = []
    for j in range(_NCHUNK):
        copies.append(
            pltpu.async_copy(
                table_hbm.at[idx_v.at[j]],
                rows_v.at[pl.ds(j * _CHUNK, _CHUNK)],
                sem,
            )
        )
    for c in copies:
        c.wait()
    pltpu.sync_copy(rows_v, out_hbm.at[pl.ds(base, _B_PER_W)])


def kernel(x, table):
    idx = x.astype(jnp.int32).reshape(_NW, _NCHUNK, _CHUNK)
    return _embed_gather(idx, table)


# speedup vs baseline: 4.1133x; 4.1133x over previous
import functools

import jax
import jax.numpy as jnp
from jax import lax
from jax.experimental import pallas as pl
from jax.experimental.pallas import tpu as pltpu
from jax.experimental.pallas import tpu_sc as plsc

_info = plsc.get_sparse_core_info()
_NC, _NS = _info.num_cores, _info.num_subcores
_NW = _NC * _NS

_BATCH = 16384
_DIM = 32
_B_PER_W = _BATCH // _NW
_G = 16
_NGROUP = _B_PER_W // _G

_mesh = plsc.VectorSubcoreMesh(core_axis_name="c", subcore_axis_name="s")


@functools.partial(
    pl.kernel,
    mesh=_mesh,
    out_type=jax.ShapeDtypeStruct((_DIM, _BATCH), jnp.float32),
    scratch_types=[
        pltpu.VMEM((_B_PER_W,), jnp.int32),
        pltpu.VMEM((_G, _DIM, 128), jnp.float32),
        pltpu.VMEM((_DIM, _B_PER_W), jnp.float32),
        pltpu.SemaphoreType.DMA((_G,)),
    ],
    compiler_params=pltpu.CompilerParams(needs_layout_passes=False),
)
def _embed_gather(idx_hbm, table_hbm, out_hbm, idx_v, ring, out_slab, sems):
    wid = lax.axis_index("s") * _NC + lax.axis_index("c")
    base = wid * _B_PER_W
    pltpu.sync_copy(idx_hbm.at[pl.ds(base, _B_PER_W)], idx_v)

    rows_lo = lax.iota(jnp.int32, 16)
    rows_hi = rows_lo + 16
    zeros16 = jnp.zeros((16,), jnp.int32)

    def fire(xi, slot):
        col = pl.multiple_of((xi >> 7) * 128, 128)
        pltpu.async_copy(
            table_hbm.at[:, pl.ds(col, 128)],
            ring.at[slot],
            sems.at[slot],
        )

    iv0 = idx_v[pl.ds(0, _G)]
    for j in range(_G):
        fire(iv0[j], j)

    def body(g, carry):
        iv = idx_v[pl.ds(g * _G, _G)]
        g_next = jnp.minimum(g + 1, _NGROUP - 1)
        iv_next = idx_v[pl.ds(g_next * _G, _G)]
        for j in range(_G):
            pltpu.make_async_copy(
                table_hbm.at[:, pl.ds(0, 128)], ring.at[j], sems.at[j]
            ).wait()
            lane = zeros16 + (iv[j] & 127)
            rvec = zeros16 + (g * _G + j)
            lo = plsc.load_gather(ring.at[j], [rows_lo, lane])
            plsc.store_scatter(out_slab, [rows_lo, rvec], lo)
            hi = plsc.load_gather(ring.at[j], [rows_hi, lane])
            plsc.store_scatter(out_slab, [rows_hi, rvec], hi)

            @pl.when(g + 1 < _NGROUP)
            def _():
                fire(iv_next[j], j)

        return carry

    lax.fori_loop(0, _NGROUP, body, 0)
    pltpu.sync_copy(out_slab, out_hbm.at[:, pl.ds(base, _B_PER_W)])


def kernel(x, table):
    out_t = _embed_gather(x.astype(jnp.int32), table.T)
    return out_t.T
